# baseline (speedup 1.0000x reference)
import functools

import jax
import jax.numpy as jnp
from jax import lax
from jax.experimental import pallas as pl
from jax.experimental.pallas import tpu as pltpu
from jax.experimental.pallas import tpu_sc as plsc

_K = 4
_H = 0.03
_H2 = _H * _H
_B = 8
_N = 2048
_LANES = 16
_NBLK = _N // _LANES
_NW = 32

_SB = 4
_W_PER_BATCH = _NW // max(_SB, 1)
_ROWS_PER_W = max(_SB, 1) * _N // _NW

_TB = _B - _SB
_RBLK = 256
_NEG = -30.0


def _sc_topk(soa):
    mesh = plsc.VectorSubcoreMesh(core_axis_name="c", subcore_axis_name="s")

    @functools.partial(
        pl.kernel,
        mesh=mesh,
        out_type=jax.ShapeDtypeStruct((_SB * _N, _LANES), jnp.float32),
        scratch_types=[
            pltpu.VMEM((4 * _N,), jnp.float32),
            pltpu.VMEM((_ROWS_PER_W, _LANES), jnp.float32),
        ],
        compiler_params=pltpu.CompilerParams(needs_layout_passes=False),
    )
    def body(soa_hbm, out_hbm, xyz, best_buf):
        wid = lax.axis_index("s") * 2 + lax.axis_index("c")
        b = wid // _W_PER_BATCH
        n0 = (wid % _W_PER_BATCH) * _ROWS_PER_W
        pltpu.sync_copy(soa_hbm.at[b], xyz)

        zeros16 = jnp.zeros((_LANES,), jnp.float32)
        unroll = 8

        def compute_p(c, row):
            px2, py2, pz2, hv = row
            cx = xyz[pl.ds(c * _LANES, _LANES)]
            cy = xyz[pl.ds(_N + c * _LANES, _LANES)]
            cz = xyz[pl.ds(2 * _N + c * _LANES, _LANES)]
            sqc = xyz[pl.ds(3 * _N + c * _LANES, _LANES)]
            return (cx * px2 + cy * py2 + cz * pz2) + (hv - sqc)

        def grp_body(g, _):
            r0 = g * _LANES
            rx = xyz[pl.ds(n0 + r0, _LANES)]
            ry = xyz[pl.ds(_N + n0 + r0, _LANES)]
            rz = xyz[pl.ds(2 * _N + n0 + r0, _LANES)]
            rsq = xyz[pl.ds(3 * _N + n0 + r0, _LANES)]
            hsq = _H2 - rsq
            for l in range(_LANES):
                row = (jnp.full((_LANES,), rx[l] * 2.0),
                       jnp.full((_LANES,), ry[l] * 2.0),
                       jnp.full((_LANES,), rz[l] * 2.0),
                       jnp.full((_LANES,), hsq[l]))

                def blk(c4, carry, row=row):
                    v1, v2, v3 = carry
                    for u in range(unroll):
                        p = compute_p(c4 * unroll + u, row)
                        b1 = jnp.minimum(v1, p)
                        v1 = jnp.maximum(v1, p)
                        b2 = jnp.minimum(v2, b1)
                        v2 = jnp.maximum(v2, b1)
                        v3 = jnp.maximum(v3, b2)
                    return (v1, v2, v3)

                v1, v2, v3 = lax.fori_loop(
                    0, _NBLK // unroll, blk, (zeros16, zeros16, zeros16))

                def exact_scan(_, row=row):
                    def blk2(c, best):
                        ps = lax.sort(compute_p(c, row))
                        return lax.sort(jnp.maximum(best, lax.rev(ps, (0,))))
                    return lax.fori_loop(0, _NBLK, blk2, zeros16)

                def fast(_, v1=v1, v2=v2):
                    s1 = lax.sort(v1)
                    s2 = lax.sort(v2)
                    return lax.sort(jnp.maximum(s1, lax.rev(s2, (0,))))

                best = lax.cond(jnp.max(v3, axis=0) > 0.0,
                                exact_scan, fast, 0)
                best_buf[r0 + l, :] = best
            return 0

        lax.fori_loop(0, _ROWS_PER_W // _LANES, grp_body, 0)
        pltpu.sync_copy(best_buf,
                        out_hbm.at[pl.ds(wid * _ROWS_PER_W, _ROWS_PER_W)])

    return body(soa)


def _tc_topk(xb, yb, zb, sqb):
    nrb = _N // _RBLK

    def body(xr, yr, zr, sr, xc, yc, zc, sc, out_ref):
        rx2 = 2.0 * xr[0, 0][:, None]
        ry2 = 2.0 * yr[0, 0][:, None]
        rz2 = 2.0 * zr[0, 0][:, None]
        hr = (_H2 - sr[0, 0])[:, None]
        cx = xc[0, 0][None, :]
        cy = yc[0, 0][None, :]
        cz = zc[0, 0][None, :]
        sqc = sc[0, 0][None, :]
        p = (cx * rx2 + cy * ry2 + cz * rz2) + (hr - sqc)
        iota = lax.broadcasted_iota(jnp.int32, (_RBLK, _N), 1)
        tops = []
        for _ in range(_K + 1):
            m = jnp.max(p, axis=1)
            j = jnp.min(jnp.where(p == m[:, None], iota, _N), axis=1)
            p = jnp.where(iota == j[:, None], _NEG, p)
            tops.append(m[:, None])
        pad = jnp.full((_RBLK, _LANES - _K - 1), _NEG, jnp.float32)
        out_ref[0] = jnp.concatenate([pad] + tops[::-1], axis=1)

    rows = lambda a: a.reshape(_TB * nrb, 1, _RBLK)
    cands = lambda a: a.reshape(_TB, 1, _N)
    row_spec = pl.BlockSpec((1, 1, _RBLK), lambda b, r: (b * nrb + r, 0, 0))
    cand_spec = pl.BlockSpec((1, 1, _N), lambda b, r: (b, 0, 0))
    out = pl.pallas_call(
        body,
        grid=(_TB, nrb),
        in_specs=[row_spec, row_spec, row_spec, row_spec,
                  cand_spec, cand_spec, cand_spec, cand_spec],
        out_specs=pl.BlockSpec(
            (1, _RBLK, _LANES), lambda b, r: (b * nrb + r, 0, 0)),
        out_shape=jax.ShapeDtypeStruct((_TB * nrb, _RBLK, _LANES),
                                       jnp.float32),
        compiler_params=pltpu.CompilerParams(
            dimension_semantics=("arbitrary", "arbitrary")),
    )(rows(xb), rows(yb), rows(zb), rows(sqb),
      cands(xb), cands(yb), cands(zb), cands(sqb))
    return out


def _tc_loss(p_sc, p_tc):
    def fsum(p):
        lane = lax.broadcasted_iota(jnp.int32, p.shape, p.ndim - 1)
        keep = (lane >= _LANES - 1 - _K) & (lane <= _LANES - 2)
        d2 = jnp.maximum(_H2 - p, 0.0)
        dist = jnp.sqrt(d2 + 1e-12)
        w = jnp.exp(-d2 / _H2)
        f = jnp.maximum(_H - dist, 0.0) * w
        return jnp.sum(jnp.where(keep, f, 0.0))

    def body(a_ref, b_ref, out_ref):
        out_ref[0, 0] = (fsum(a_ref[...]) + fsum(b_ref[...])) / (_B * _N * _K)

    return pl.pallas_call(
        body,
        out_shape=jax.ShapeDtypeStruct((1, 1), jnp.float32),
        out_specs=pl.BlockSpec(memory_space=pltpu.SMEM),
    )(p_sc, p_tc)


def _round_bf16(x):
    u = lax.bitcast_convert_type(x, jnp.uint32)
    r = (u + jnp.uint32(0x7FFF) + ((u >> 16) & jnp.uint32(1))) \
        & jnp.uint32(0xFFFF0000)
    return lax.bitcast_convert_type(r, jnp.float32)


def kernel(pcs):
    pb = _round_bf16(pcs)
    sq = jnp.sum(pcs * pcs, axis=-1)
    pbt = jnp.transpose(pb, (0, 2, 1))
    soa = jnp.concatenate(
        [pbt[:_SB].reshape(_SB, 3 * _N), sq[:_SB]], axis=1)
    p_sc = _sc_topk(soa)
    p_tc = _tc_topk(pbt[_SB:, 0], pbt[_SB:, 1], pbt[_SB:, 2], sq[_SB:])
    loss = _tc_loss(p_sc, p_tc)
    return loss[0, 0]


# speedup vs baseline: 1.1227x; 1.1227x over previous
import functools

import jax
import jax.numpy as jnp
from jax import lax
from jax.experimental import pallas as pl
from jax.experimental.pallas import tpu as pltpu
from jax.experimental.pallas import tpu_sc as plsc

_K = 4
_H = 0.03
_H2 = _H * _H
_B = 8
_N = 2048
_LANES = 16
_NBLK = _N // _LANES
_NW = 32

_SB = 4
_W_PER_BATCH = _NW // max(_SB, 1)
_ROWS_PER_W = max(_SB, 1) * _N // _NW

_TB = _B - _SB
_RBLK = 256
_NEG = -30.0


def _sc_loss_part(soa):
    mesh = plsc.VectorSubcoreMesh(core_axis_name="c", subcore_axis_name="s")

    @functools.partial(
        pl.kernel,
        mesh=mesh,
        out_type=jax.ShapeDtypeStruct((_NW, _LANES), jnp.float32),
        scratch_types=[
            pltpu.VMEM((4 * _N,), jnp.float32),
            pltpu.VMEM((_LANES,), jnp.float32),
        ],
        compiler_params=pltpu.CompilerParams(needs_layout_passes=False),
    )
    def body(soa_hbm, out_hbm, xyz, accbuf):
        wid = lax.axis_index("s") * 2 + lax.axis_index("c")
        b = wid // _W_PER_BATCH
        n0 = (wid % _W_PER_BATCH) * _ROWS_PER_W
        pltpu.sync_copy(soa_hbm.at[b], xyz)

        zeros16 = jnp.zeros((_LANES,), jnp.float32)
        lane_iota = lax.iota(jnp.int32, _LANES)
        keep = (lane_iota >= _LANES - 1 - _K) & (lane_iota <= _LANES - 2)
        unroll = 4

        def compute_p(c, row):
            px2, py2, pz2, hv = row
            cx = xyz[pl.ds(c * _LANES, _LANES)]
            cy = xyz[pl.ds(_N + c * _LANES, _LANES)]
            cz = xyz[pl.ds(2 * _N + c * _LANES, _LANES)]
            sqc = xyz[pl.ds(3 * _N + c * _LANES, _LANES)]
            return (cx * px2 + cy * py2 + cz * pz2) + (hv - sqc)

        def grp_body(g, acc):
            r0 = g * _LANES
            rx = xyz[pl.ds(n0 + r0, _LANES)]
            ry = xyz[pl.ds(_N + n0 + r0, _LANES)]
            rz = xyz[pl.ds(2 * _N + n0 + r0, _LANES)]
            rsq = xyz[pl.ds(3 * _N + n0 + r0, _LANES)]
            hsq = _H2 - rsq
            for l in range(_LANES):
                row = (jnp.full((_LANES,), rx[l] * 2.0),
                       jnp.full((_LANES,), ry[l] * 2.0),
                       jnp.full((_LANES,), rz[l] * 2.0),
                       jnp.full((_LANES,), hsq[l]))

                def blk(c4, carry, row=row):
                    v1, v2, v3 = carry
                    for u in range(unroll):
                        p = compute_p(c4 * unroll + u, row)
                        b1 = jnp.minimum(v1, p)
                        v1 = jnp.maximum(v1, p)
                        b2 = jnp.minimum(v2, b1)
                        v2 = jnp.maximum(v2, b1)
                        v3 = jnp.maximum(v3, b2)
                    return (v1, v2, v3)

                v1, v2, v3 = lax.fori_loop(
                    0, _NBLK // unroll, blk, (zeros16, zeros16, zeros16))

                def exact_scan(_, row=row):
                    def blk2(c, best):
                        ps = lax.sort(compute_p(c, row))
                        return lax.sort(jnp.maximum(best, lax.rev(ps, (0,))))
                    return lax.fori_loop(0, _NBLK, blk2, zeros16)

                def fast(_, v1=v1, v2=v2):
                    s1 = lax.sort(v1)
                    s2 = lax.sort(v2)
                    return lax.sort(jnp.maximum(s1, lax.rev(s2, (0,))))

                best = lax.cond(jnp.max(v3, axis=0) > 0.0,
                                exact_scan, fast, 0)
                d2 = jnp.maximum(_H2 - best, 0.0)
                x = d2 + 1e-12
                i = lax.bitcast_convert_type(x, jnp.int32)
                y = lax.bitcast_convert_type(
                    jnp.int32(0x5F3759DF) - (i >> 1), jnp.float32)
                y = y * (1.5 - 0.5 * x * y * y)
                y = y * (1.5 - 0.5 * x * y * y)
                y = y * (1.5 - 0.5 * x * y * y)
                dist = x * y
                f = jnp.maximum(_H - dist, 0.0) * jnp.exp(d2 * (-1.0 / _H2))
                acc = acc + jnp.where(keep, f, 0.0)
            return acc

        acc = lax.fori_loop(0, _ROWS_PER_W // _LANES, grp_body, zeros16)
        accbuf[:] = acc
        pltpu.sync_copy(accbuf, out_hbm.at[wid])

    return body(soa)


def _tc_loss_part(xb, yb, zb, sqb):
    nrb = _N // _RBLK

    def body(xr, yr, zr, sr, xc, yc, zc, sc, out_ref):
        rx2 = 2.0 * xr[0, 0][:, None]
        ry2 = 2.0 * yr[0, 0][:, None]
        rz2 = 2.0 * zr[0, 0][:, None]
        hr = (_H2 - sr[0, 0])[:, None]
        cx = xc[0, 0][None, :]
        cy = yc[0, 0][None, :]
        cz = zc[0, 0][None, :]
        sqc = sc[0, 0][None, :]
        p = (cx * rx2 + cy * ry2 + cz * rz2) + (hr - sqc)
        iota = lax.broadcasted_iota(jnp.int32, (_RBLK, _N), 1)
        ftot = jnp.float32(0.0)
        for t in range(_K + 1):
            m = jnp.max(p, axis=1)
            j = jnp.min(jnp.where(p == m[:, None], iota, _N), axis=1)
            p = jnp.where(iota == j[:, None], _NEG, p)
            if t > 0:
                d2 = jnp.maximum(_H2 - m, 0.0)
                dist = jnp.sqrt(d2 + 1e-12)
                f = jnp.maximum(_H - dist, 0.0) * jnp.exp(d2 * (-1.0 / _H2))
                ftot = ftot + jnp.sum(f)

        @pl.when((pl.program_id(0) == 0) & (pl.program_id(1) == 0))
        def _():
            out_ref[0, 0] = 0.0

        out_ref[0, 0] += ftot

    rows = lambda a: a.reshape(_TB * nrb, 1, _RBLK)
    cands = lambda a: a.reshape(_TB, 1, _N)
    row_spec = pl.BlockSpec((1, 1, _RBLK), lambda b, r: (b * nrb + r, 0, 0))
    cand_spec = pl.BlockSpec((1, 1, _N), lambda b, r: (b, 0, 0))
    return pl.pallas_call(
        body,
        grid=(_TB, nrb),
        in_specs=[row_spec, row_spec, row_spec, row_spec,
                  cand_spec, cand_spec, cand_spec, cand_spec],
        out_specs=pl.BlockSpec(memory_space=pltpu.SMEM),
        out_shape=jax.ShapeDtypeStruct((1, 1), jnp.float32),
        compiler_params=pltpu.CompilerParams(
            dimension_semantics=("arbitrary", "arbitrary")),
    )(rows(xb), rows(yb), rows(zb), rows(sqb),
      cands(xb), cands(yb), cands(zb), cands(sqb))


def _round_bf16(x):
    u = lax.bitcast_convert_type(x, jnp.uint32)
    r = (u + jnp.uint32(0x7FFF) + ((u >> 16) & jnp.uint32(1))) \
        & jnp.uint32(0xFFFF0000)
    return lax.bitcast_convert_type(r, jnp.float32)


def kernel(pcs):
    pb = _round_bf16(pcs)
    sq = jnp.sum(pcs * pcs, axis=-1)
    pbt = jnp.transpose(pb, (0, 2, 1))
    soa = jnp.concatenate(
        [pbt[:_SB].reshape(_SB, 3 * _N), sq[:_SB]], axis=1)
    sc_part = _sc_loss_part(soa)
    tc_part = _tc_loss_part(
        pbt[_SB:, 0], pbt[_SB:, 1], pbt[_SB:, 2], sq[_SB:])
    return (jnp.sum(sc_part) + tc_part[0, 0]) / (_B * _N * _K)
